# baseline (speedup 1.0000x reference)
import functools

import jax
import jax.numpy as jnp
from jax import lax
from jax.experimental import pallas as pl
from jax.experimental.pallas import tpu as pltpu
from jax.experimental.pallas import tpu_sc as plsc

M = 32
B = 16384
N_CTS = 4
N_CAT = 26
LEVELS = 100000
N_FIELDS = N_CTS + N_CAT

NC = 2
NS = 16
NW = NC * NS

PAIRS = N_FIELDS * M
PPW = PAIRS // NW
BCH = 4096
NBCH = B // BCH


def _sc_body(table_t, idx_t, i0_t, t_t, emb_t, out, row_v, idx_v, col_a,
             col_b, t_v, ec_v, sem_w):
    wid = lax.axis_index("s") * NC + lax.axis_index("c")
    cols = [col_a, col_b]

    def write_chunks(f, d, gather_chunk):
        handles = [None, None]
        for c in range(NBCH):
            col = cols[c % 2]
            if handles[c % 2] is not None:
                handles[c % 2].wait()
            gather_chunk(c, col)
            handles[c % 2] = pltpu.async_copy(
                col, out.at[f, d, pl.ds(c * BCH, BCH)], sem_w)
        for h in handles:
            if h is not None:
                h.wait()

    def pair(p, jprev):
        gp = wid * PPW + p
        f = gp // M
        d = gp % M

        @pl.when(f >= N_CTS)
        def _cat():
            j = f - N_CTS

            @pl.when(j != jprev)
            def _stage_idx():
                pltpu.sync_copy(idx_t.at[j], idx_v)

            pltpu.sync_copy(table_t.at[j, d], row_v)

            def gather_chunk(c, col):
                def vec(k, carry3):
                    g = plsc.load_gather(
                        row_v, [idx_v[pl.ds(c * BCH + k * 16, 16)]])
                    col[pl.ds(k * 16, 16)] = g
                    return carry3

                lax.fori_loop(0, BCH // 16, vec, 0, unroll=8)

            write_chunks(f, d, gather_chunk)

        @pl.when(f < N_CTS)
        def _cts():
            pltpu.sync_copy(emb_t.at[f, d], ec_v)
            pltpu.sync_copy(i0_t.at[f], idx_v)

            def gather_chunk(c, col):
                pltpu.sync_copy(t_t.at[f, pl.ds(c * BCH, BCH)], t_v)

                def vec(k, carry3):
                    i016 = idx_v[pl.ds(c * BCH + k * 16, 16)]
                    t16 = t_v[pl.ds(k * 16, 16)]
                    e0 = plsc.load_gather(ec_v, [i016])
                    e1 = plsc.load_gather(ec_v, [i016 + 1])
                    col[pl.ds(k * 16, 16)] = e0 + t16 * (e1 - e0)
                    return carry3

                lax.fori_loop(0, BCH // 16, vec, 0, unroll=8)

            write_chunks(f, d, gather_chunk)

        return jnp.where(f >= N_CTS, f - N_CTS, -1)

    lax.fori_loop(0, PPW, pair, -1)


@functools.partial(
    pl.kernel,
    out_type=jax.ShapeDtypeStruct((N_FIELDS, M, B), jnp.float32),
    mesh=plsc.VectorSubcoreMesh(core_axis_name="c", subcore_axis_name="s"),
    compiler_params=pltpu.CompilerParams(
        use_tc_tiling_on_sc=True, needs_layout_passes=False),
    scratch_types=[
        pltpu.VMEM((LEVELS,), jnp.float32),
        pltpu.VMEM((B,), jnp.int32),
        pltpu.VMEM((BCH,), jnp.float32),
        pltpu.VMEM((BCH,), jnp.float32),
        pltpu.VMEM((BCH,), jnp.float32),
        pltpu.VMEM((M,), jnp.float32),
        pltpu.SemaphoreType.DMA,
    ],
)
def _sc_cols(table_t, idx_t, i0_t, t_t, emb_t, out, row_v, idx_v, col_a,
             col_b, t_v, ec_v, sem_w):
    _sc_body(table_t, idx_t, i0_t, t_t, emb_t, out, row_v, idx_v, col_a,
             col_b, t_v, ec_v, sem_w)


BC = 16384


def _prep_body(xf_ref, lm_ref, i0_ref, t_ref):
    xv = xf_ref[0, 0, :]
    lm = lm_ref[0, 0]
    indx = jnp.sum((lm[None, :] < xv[:, None]).astype(jnp.int32), axis=1)
    indx = jnp.clip(indx, 1, M - 1)
    cols = lax.broadcasted_iota(jnp.int32, (B, M), 1)
    oh1 = (cols == indx[:, None]).astype(jnp.float32)
    oh0 = (cols == indx[:, None] - 1).astype(jnp.float32)
    lm1 = jnp.sum(oh1 * lm[None, :], axis=1)
    lm0 = jnp.sum(oh0 * lm[None, :], axis=1)
    i0_ref[0, 0, :] = indx - 1
    t_ref[0, 0, :] = (xv - lm0) / (lm1 - lm0)


_prep_call = pl.pallas_call(
    _prep_body,
    grid=(N_CTS,),
    in_specs=[
        pl.BlockSpec((1, 1, B), lambda i: (i, 0, 0)),
        pl.BlockSpec((1, 1, M), lambda i: (i, 0, 0)),
    ],
    out_specs=[
        pl.BlockSpec((1, 1, B), lambda i: (i, 0, 0)),
        pl.BlockSpec((1, 1, B), lambda i: (i, 0, 0)),
    ],
    out_shape=[
        jax.ShapeDtypeStruct((N_CTS, 1, B), jnp.int32),
        jax.ShapeDtypeStruct((N_CTS, 1, B), jnp.float32),
    ],
)


def kernel(x, landmarks, cts_emb_landmarks, cat_tables):
    xft3 = x[:, :N_CTS].astype(jnp.float32).T.reshape(N_CTS, 1, B)
    lm3 = landmarks.reshape(N_CTS, 1, M)
    i03, t3 = _prep_call(xft3, lm3)

    table_t = cat_tables.transpose(0, 2, 1)
    emb_t = cts_emb_landmarks.transpose(0, 2, 1)
    idx_t = x[:, N_CTS:].T

    out_t = _sc_cols(table_t, idx_t, i03.reshape(N_CTS, B),
                     t3.reshape(N_CTS, B), emb_t)
    return out_t.transpose(2, 0, 1)
